# Initial kernel scaffold; baseline (speedup 1.0000x reference)
import jax
import jax.numpy as jnp
from jax.experimental import pallas as pl


def kernel(patch_embs, edge_index, edge_attr, W_spline, W_root, b_root, bn1_g, bn1_b, bn2_g, bn2_b, bn3_g, bn3_b, Wq2, Wk2, Wv2, Wo2, proj2, Wq3, Wk3, Wv3, Wo3, proj3):
    raise NotImplementedError("write your pallas kernel here")


# speedup vs baseline: 51.8632x; 51.8632x over previous
import functools
import jax
import jax.numpy as jnp
from jax import lax
from jax.experimental import pallas as pl
from jax.experimental.pallas import tpu as pltpu, tpu_sc as plsc

N = 50000
E = 800000
D = 32
H = 8
DH = 4
M = 16
TAU = 1.0

NC = 2
NS = 16
NW = NC * NS
EPW = E // NW

CHS = 64
NFS = EPW // CHS
CTS = EPW - NFS * CHS

CHR = 128
NFR = EPW // CHR
CTR = EPW - NFR * CHR

PZ = 200
NPZ = N // PZ
NPFULL = NPZ // NS
NPREM = NPZ - NPFULL * NS

BN = 2000
NB = N // BN

_f32 = jnp.float32


def _dot(a, b):
    return jax.lax.dot_general(
        a, b, (((a.ndim - 1,), (0,)), ((), ())),
        precision=jax.lax.Precision.HIGHEST, preferred_element_type=_f32)


def _lrelu(x):
    return jnp.where(x >= 0, x, 0.01 * x)


def _prep_body(x_ref, wc_ref, wr_ref, br_ref, p_ref, xr_ref):
    xb = x_ref[...]
    p_ref[...] = _dot(xb, wc_ref[...])
    xr_ref[...] = _dot(xb, wr_ref[...]) + br_ref[...]


def _tck_prep(x, wcat, wr, br):
    return pl.pallas_call(
        _prep_body,
        grid=(NB,),
        in_specs=[
            pl.BlockSpec((BN, D), lambda i: (i, 0)),
            pl.BlockSpec((D, 8 * D), lambda i: (0, 0)),
            pl.BlockSpec((D, D), lambda i: (0, 0)),
            pl.BlockSpec((1, D), lambda i: (0, 0)),
        ],
        out_specs=[
            pl.BlockSpec((BN, 8 * D), lambda i: (i, 0)),
            pl.BlockSpec((BN, D), lambda i: (i, 0)),
        ],
        out_shape=[
            jax.ShapeDtypeStruct((N, 8 * D), _f32),
            jax.ShapeDtypeStruct((N, D), _f32),
        ],
    )(x, wcat, wr, br)


def _sc_spline_body(p_hbm, src_hbm, dst_hbm, u0_hbm, u1_hbm, u2_hbm,
                    z32_hbm,
                    agg_out,
                    idxs, idxd, rows, outb,
                    idxs_t, idxd_t,
                    u0v, u1v, u2v,
                    zbuf, agg_sh):
    cid = lax.axis_index("c")
    sid = lax.axis_index("s")
    wid = sid * NC + cid

    pltpu.sync_copy(z32_hbm, zbuf)

    def _zero_piece(r0):
        r0 = pl.multiple_of(r0, 8)
        pltpu.sync_copy(zbuf, agg_sh.at[pl.ds(r0, PZ)])

    for r in range(NPFULL):
        _zero_piece((sid + NS * r) * PZ)

    @pl.when(sid < NPREM)
    def _():
        _zero_piece((sid + NS * NPFULL) * PZ)

    plsc.subcore_barrier()

    ebase = wid * EPW

    def do_chunk(base, cc, b_idxs, b_idxd):
        base = pl.multiple_of(base, 8)
        pltpu.sync_copy(src_hbm.at[pl.ds(base, cc)], b_idxs)
        pltpu.sync_copy(dst_hbm.at[pl.ds(base, cc)], b_idxd)
        pltpu.sync_copy(u0_hbm.at[pl.ds(base, cc)], u0v.at[pl.ds(0, cc)])
        pltpu.sync_copy(u1_hbm.at[pl.ds(base, cc)], u1v.at[pl.ds(0, cc)])
        pltpu.sync_copy(u2_hbm.at[pl.ds(base, cc)], u2v.at[pl.ds(0, cc)])
        pltpu.sync_copy(p_hbm.at[b_idxs], rows.at[pl.ds(0, cc)])

        groups = []
        off = 0
        while off < cc:
            groups.append((off, min(16, cc - off)))
            off += 16
        for goff, nj in groups:
            u0 = u0v[pl.ds(goff, 16)]
            u1 = u1v[pl.ds(goff, 16)]
            u2 = u2v[pl.ds(goff, 16)]
            a0 = 1.0 - u0
            a1 = 1.0 - u1
            a2 = 1.0 - u2
            t0 = a0 * a1
            t1 = u0 * a1
            t2 = a0 * u1
            t3 = u0 * u1
            bs = (t0 * a2, t1 * a2, t2 * a2, t3 * a2,
                  t0 * u2, t1 * u2, t2 * u2, t3 * u2)
            for j in range(nj):
                e = goff + j
                lo = bs[0][j] * rows[e, pl.ds(0, 16)]
                hi = bs[0][j] * rows[e, pl.ds(16, 16)]
                for s in range(1, 8):
                    lo = lo + bs[s][j] * rows[e, pl.ds(s * 32, 16)]
                    hi = hi + bs[s][j] * rows[e, pl.ds(s * 32 + 16, 16)]
                outb[e, pl.ds(0, 16)] = lo
                outb[e, pl.ds(16, 16)] = hi

        pltpu.sync_copy(outb.at[pl.ds(0, cc)], agg_sh.at[b_idxd], add=True)

    def cbody(c, _):
        do_chunk(ebase + c * CHS, CHS, idxs, idxd)
        return 0

    lax.fori_loop(0, NFS, cbody, 0)
    do_chunk(ebase + NFS * CHS, CTS, idxs_t, idxd_t)

    plsc.subcore_barrier()

    def _drain_piece(r0):
        r0 = pl.multiple_of(r0, 8)
        pltpu.sync_copy(agg_sh.at[pl.ds(r0, PZ)], zbuf)
        pltpu.sync_copy(zbuf, agg_out.at[cid, pl.ds(r0, PZ)])

    for r in range(NPFULL):
        _drain_piece((sid + NS * r) * PZ)

    @pl.when(sid < NPREM)
    def _():
        _drain_piece((sid + NS * NPFULL) * PZ)


def _sc_spline(p, src, dst, u0, u1, u2, z32):
    mesh = plsc.VectorSubcoreMesh(
        core_axis_name="c", subcore_axis_name="s",
        num_cores=NC, num_subcores=NS)
    f = pl.kernel(
        _sc_spline_body,
        out_type=[
            jax.ShapeDtypeStruct((NC, N, D), _f32),
        ],
        mesh=mesh,
        scratch_types=[
            pltpu.VMEM((CHS,), jnp.int32),
            pltpu.VMEM((CHS,), jnp.int32),
            pltpu.VMEM((CHS, 8 * D), _f32),
            pltpu.VMEM((CHS, D), _f32),
            pltpu.VMEM((CTS,), jnp.int32),
            pltpu.VMEM((CTS,), jnp.int32),
            pltpu.VMEM((CHS + 16,), _f32),
            pltpu.VMEM((CHS + 16,), _f32),
            pltpu.VMEM((CHS + 16,), _f32),
            pltpu.VMEM((PZ, D), _f32),
            pltpu.VMEM_SHARED((N, D), _f32),
        ],
        compiler_params=pltpu.CompilerParams(use_tc_tiling_on_sc=False),
    )
    return f(p, src, dst, u0, u1, u2, z32)[0]


def _sc_deg_body(dst_hbm, ones_hbm, z16_hbm,
                 deg_out,
                 idxd, idxd_t, onesb, zbuf, deg_sh):
    cid = lax.axis_index("c")
    sid = lax.axis_index("s")
    wid = sid * NC + cid

    pltpu.sync_copy(z16_hbm, zbuf)
    pltpu.sync_copy(ones_hbm, onesb)

    def _zero_piece(r0):
        r0 = pl.multiple_of(r0, 8)
        pltpu.sync_copy(zbuf, deg_sh.at[pl.ds(r0, PZ)])

    for r in range(NPFULL):
        _zero_piece((sid + NS * r) * PZ)

    @pl.when(sid < NPREM)
    def _():
        _zero_piece((sid + NS * NPFULL) * PZ)

    plsc.subcore_barrier()

    ebase = wid * EPW

    def do_chunk(base, cc, b_idxd):
        base = pl.multiple_of(base, 8)
        pltpu.sync_copy(dst_hbm.at[pl.ds(base, cc)], b_idxd)
        pltpu.sync_copy(onesb.at[pl.ds(0, cc)], deg_sh.at[b_idxd], add=True)

    def cbody(c, _):
        do_chunk(ebase + c * CHR, CHR, idxd)
        return 0

    lax.fori_loop(0, NFR, cbody, 0)
    do_chunk(ebase + NFR * CHR, CTR, idxd_t)

    plsc.subcore_barrier()

    def _drain_piece(r0):
        r0 = pl.multiple_of(r0, 8)
        pltpu.sync_copy(deg_sh.at[pl.ds(r0, PZ)], zbuf)
        pltpu.sync_copy(zbuf, deg_out.at[cid, pl.ds(r0, PZ)])

    for r in range(NPFULL):
        _drain_piece((sid + NS * r) * PZ)

    @pl.when(sid < NPREM)
    def _():
        _drain_piece((sid + NS * NPFULL) * PZ)


def _sc_deg(dst, ones16, z16):
    mesh = plsc.VectorSubcoreMesh(
        core_axis_name="c", subcore_axis_name="s",
        num_cores=NC, num_subcores=NS)
    f = pl.kernel(
        _sc_deg_body,
        out_type=[jax.ShapeDtypeStruct((NC, N, 16), _f32)],
        mesh=mesh,
        scratch_types=[
            pltpu.VMEM((CHR,), jnp.int32),
            pltpu.VMEM((CTR,), jnp.int32),
            pltpu.VMEM((CHR, 16), _f32),
            pltpu.VMEM((PZ, 16), _f32),
            pltpu.VMEM_SHARED((N, 16), _f32),
        ],
        compiler_params=pltpu.CompilerParams(use_tc_tiling_on_sc=False),
    )
    return f(dst, ones16, z16)[0]


def _sc_rel_body(v_hbm, src_hbm, dst_hbm, z32_hbm,
                 rel_out,
                 idxs, idxd, rows,
                 idxs_t, idxd_t,
                 zbuf, rel_sh):
    cid = lax.axis_index("c")
    sid = lax.axis_index("s")
    wid = sid * NC + cid

    pltpu.sync_copy(z32_hbm, zbuf)

    def _zero_piece(r0):
        r0 = pl.multiple_of(r0, 8)
        pltpu.sync_copy(zbuf, rel_sh.at[pl.ds(r0, PZ)])

    for r in range(NPFULL):
        _zero_piece((sid + NS * r) * PZ)

    @pl.when(sid < NPREM)
    def _():
        _zero_piece((sid + NS * NPFULL) * PZ)

    plsc.subcore_barrier()

    ebase = wid * EPW

    def do_chunk(base, cc, b_idxs, b_idxd):
        base = pl.multiple_of(base, 8)
        pltpu.sync_copy(src_hbm.at[pl.ds(base, cc)], b_idxs)
        pltpu.sync_copy(dst_hbm.at[pl.ds(base, cc)], b_idxd)
        pltpu.sync_copy(v_hbm.at[b_idxs], rows.at[pl.ds(0, cc)])
        pltpu.sync_copy(rows.at[pl.ds(0, cc)], rel_sh.at[b_idxd], add=True)

    def cbody(c, _):
        do_chunk(ebase + c * CHR, CHR, idxs, idxd)
        return 0

    lax.fori_loop(0, NFR, cbody, 0)
    do_chunk(ebase + NFR * CHR, CTR, idxs_t, idxd_t)

    plsc.subcore_barrier()

    def _drain_piece(r0):
        r0 = pl.multiple_of(r0, 8)
        pltpu.sync_copy(rel_sh.at[pl.ds(r0, PZ)], zbuf)
        pltpu.sync_copy(zbuf, rel_out.at[cid, pl.ds(r0, PZ)])

    for r in range(NPFULL):
        _drain_piece((sid + NS * r) * PZ)

    @pl.when(sid < NPREM)
    def _():
        _drain_piece((sid + NS * NPFULL) * PZ)


def _sc_rel(v, src, dst, z32):
    mesh = plsc.VectorSubcoreMesh(
        core_axis_name="c", subcore_axis_name="s",
        num_cores=NC, num_subcores=NS)
    f = pl.kernel(
        _sc_rel_body,
        out_type=[jax.ShapeDtypeStruct((NC, N, D), _f32)],
        mesh=mesh,
        scratch_types=[
            pltpu.VMEM((CHR,), jnp.int32),
            pltpu.VMEM((CHR,), jnp.int32),
            pltpu.VMEM((CHR, D), _f32),
            pltpu.VMEM((CTR,), jnp.int32),
            pltpu.VMEM((CTR,), jnp.int32),
            pltpu.VMEM((PZ, D), _f32),
            pltpu.VMEM_SHARED((N, D), _f32),
        ],
        compiler_params=pltpu.CompilerParams(use_tc_tiling_on_sc=False),
    )
    return f(v, src, dst, z32)[0]


def _post1_body(agg_ref, deg_ref, xr_ref, g_ref, b_ref, x1_ref,
                ts_ref, acc_ref):
    p = pl.program_id(0)
    i = pl.program_id(1)

    @pl.when(p == 0)
    def _():
        degc = jnp.maximum(deg_ref[0, :, 0:1] + deg_ref[1, :, 0:1], 1.0)
        t = (agg_ref[0] + agg_ref[1]) / degc + xr_ref[...]
        t = _lrelu(t)

        @pl.when(i == 0)
        def _():
            acc_ref[...] = jnp.zeros_like(acc_ref)

        acc_ref[0:1, :] += jnp.sum(t, axis=0, keepdims=True)
        acc_ref[1:2, :] += jnp.sum(t * t, axis=0, keepdims=True)
        ts_ref[pl.ds(i * BN, BN), :] = t
        x1_ref[...] = t

    @pl.when(p == 1)
    def _():
        mu = acc_ref[0:1, :] / N
        var = acc_ref[1:2, :] / N - mu * mu
        inv = lax.rsqrt(var + 1e-5)
        x1_ref[...] = ((ts_ref[pl.ds(i * BN, BN), :] - mu) * inv
                       * g_ref[...] + b_ref[...])


def _tck_post1(agg2, deg2, xr, g, b):
    return pl.pallas_call(
        _post1_body,
        grid=(2, NB),
        in_specs=[
            pl.BlockSpec((NC, BN, D), lambda p, i: (0, i, 0)),
            pl.BlockSpec((NC, BN, 16), lambda p, i: (0, i, 0)),
            pl.BlockSpec((BN, D), lambda p, i: (i, 0)),
            pl.BlockSpec((1, D), lambda p, i: (0, 0)),
            pl.BlockSpec((1, D), lambda p, i: (0, 0)),
        ],
        out_specs=pl.BlockSpec((BN, D), lambda p, i: (i, 0)),
        out_shape=jax.ShapeDtypeStruct((N, D), _f32),
        scratch_shapes=[
            pltpu.VMEM((N, D), _f32),
            pltpu.VMEM((2, D), _f32),
        ],
    )(agg2, deg2, xr, g, b)


def _qkv_body(x_ref, wq_ref, wk_ref, wv_ref, pexp_ref, smask_ref,
              phiq_ref, v_ref, kv_ref, sp_ref, kvacc, spacc):
    i = pl.program_id(0)
    xb = x_ref[...]
    scale = 1.0 / (DH ** 0.25 * TAU)
    q = _dot(xb, wq_ref[...]) * scale
    k = _dot(xb, wk_ref[...]) * scale
    v = _dot(xb, wv_ref[...])
    mscale = 1.0 / (M ** 0.5)
    phq = jnp.exp(_dot(q, pexp_ref[...]) - 0.5 * _dot(q * q, smask_ref[...]))
    phk = jnp.exp(_dot(k, pexp_ref[...]) - 0.5 * _dot(k * k, smask_ref[...]))
    phq = phq * mscale
    phk = phk * mscale
    phiq_ref[...] = phq
    v_ref[...] = v

    @pl.when(i == 0)
    def _():
        kvacc[...] = jnp.zeros_like(kvacc)
        spacc[...] = jnp.zeros_like(spacc)

    kvacc[...] += jax.lax.dot_general(
        phk, v, (((0,), (0,)), ((), ())),
        precision=jax.lax.Precision.HIGHEST, preferred_element_type=_f32)
    spacc[...] += jnp.sum(phk, axis=0, keepdims=True)

    @pl.when(i == NB - 1)
    def _():
        kv_ref[...] = kvacc[...]
        sp_ref[...] = spacc[...]


def _tck_qkv(x, wq, wk, wv, pexp, smask):
    return pl.pallas_call(
        _qkv_body,
        grid=(NB,),
        in_specs=[
            pl.BlockSpec((BN, D), lambda i: (i, 0)),
            pl.BlockSpec((D, D), lambda i: (0, 0)),
            pl.BlockSpec((D, D), lambda i: (0, 0)),
            pl.BlockSpec((D, D), lambda i: (0, 0)),
            pl.BlockSpec((D, H * M), lambda i: (0, 0)),
            pl.BlockSpec((D, H * M), lambda i: (0, 0)),
        ],
        out_specs=[
            pl.BlockSpec((BN, H * M), lambda i: (i, 0)),
            pl.BlockSpec((BN, D), lambda i: (i, 0)),
            pl.BlockSpec((H * M, D), lambda i: (0, 0)),
            pl.BlockSpec((1, H * M), lambda i: (0, 0)),
        ],
        out_shape=[
            jax.ShapeDtypeStruct((N, H * M), _f32),
            jax.ShapeDtypeStruct((N, D), _f32),
            jax.ShapeDtypeStruct((H * M, D), _f32),
            jax.ShapeDtypeStruct((1, H * M), _f32),
        ],
        scratch_shapes=[
            pltpu.VMEM((H * M, D), _f32),
            pltpu.VMEM((1, H * M), _f32),
        ],
    )(x, wq, wk, wv, pexp, smask)


def _attn_body(phiq_ref, kv_ref, sp_ref, bmask_ref, attn_ref):
    ph = phiq_ref[...]
    bm = bmask_ref[...]
    num = _dot(ph, kv_ref[...] * bm)
    den = _dot(ph * sp_ref[...], bm) + 1e-6
    attn_ref[...] = num / den


def _tck_attn(phiq, kv, sp, bmask):
    return pl.pallas_call(
        _attn_body,
        grid=(NB,),
        in_specs=[
            pl.BlockSpec((BN, H * M), lambda i: (i, 0)),
            pl.BlockSpec((H * M, D), lambda i: (0, 0)),
            pl.BlockSpec((1, H * M), lambda i: (0, 0)),
            pl.BlockSpec((H * M, D), lambda i: (0, 0)),
        ],
        out_specs=pl.BlockSpec((BN, D), lambda i: (i, 0)),
        out_shape=jax.ShapeDtypeStruct((N, D), _f32),
    )(phiq, kv, sp, bmask)


def _make_post_body(use_lrelu):
    def _post_body(attn_ref, rel_ref, deg_ref, wo_ref, g_ref, b_ref,
                   xo_ref, ts_ref, acc_ref):
        p = pl.program_id(0)
        i = pl.program_id(1)

        @pl.when(p == 0)
        def _():
            degc = jnp.maximum(deg_ref[0, :, 0:1] + deg_ref[1, :, 0:1], 1.0)
            relc = (rel_ref[0] + rel_ref[1]) / degc
            t = _dot(attn_ref[...] + relc, wo_ref[...])
            if use_lrelu:
                t = _lrelu(t)

            @pl.when(i == 0)
            def _():
                acc_ref[...] = jnp.zeros_like(acc_ref)

            acc_ref[0:1, :] += jnp.sum(t, axis=0, keepdims=True)
            acc_ref[1:2, :] += jnp.sum(t * t, axis=0, keepdims=True)
            ts_ref[pl.ds(i * BN, BN), :] = t
            xo_ref[...] = t

        @pl.when(p == 1)
        def _():
            mu = acc_ref[0:1, :] / N
            var = acc_ref[1:2, :] / N - mu * mu
            inv = lax.rsqrt(var + 1e-5)
            xo_ref[...] = ((ts_ref[pl.ds(i * BN, BN), :] - mu) * inv
                           * g_ref[...] + b_ref[...])

    return _post_body


def _tck_post(attn, rel2, deg2, wo, g, b, use_lrelu):
    return pl.pallas_call(
        _make_post_body(use_lrelu),
        grid=(2, NB),
        in_specs=[
            pl.BlockSpec((BN, D), lambda p, i: (i, 0)),
            pl.BlockSpec((NC, BN, D), lambda p, i: (0, i, 0)),
            pl.BlockSpec((NC, BN, 16), lambda p, i: (0, i, 0)),
            pl.BlockSpec((D, D), lambda p, i: (0, 0)),
            pl.BlockSpec((1, D), lambda p, i: (0, 0)),
            pl.BlockSpec((1, D), lambda p, i: (0, 0)),
        ],
        out_specs=pl.BlockSpec((BN, D), lambda p, i: (i, 0)),
        out_shape=jax.ShapeDtypeStruct((N, D), _f32),
        scratch_shapes=[
            pltpu.VMEM((N, D), _f32),
            pltpu.VMEM((2, D), _f32),
        ],
    )(attn, rel2, deg2, wo, g, b)


def kernel(patch_embs, edge_index, edge_attr, W_spline, W_root, b_root,
           bn1_g, bn1_b, bn2_g, bn2_b, bn3_g, bn3_b,
           Wq2, Wk2, Wv2, Wo2, proj2, Wq3, Wk3, Wv3, Wo3, proj3):
    x = patch_embs
    src = edge_index[0].astype(jnp.int32)
    dst = edge_index[1].astype(jnp.int32)
    u0 = edge_attr[:, 0]
    u1 = edge_attr[:, 1]
    u2 = edge_attr[:, 2]

    wcat = jnp.transpose(W_spline, (1, 0, 2)).reshape(D, 8 * D)
    eye8 = jnp.eye(H, dtype=_f32)
    smask = jnp.kron(eye8, jnp.ones((DH, M), _f32))
    bmask = jnp.kron(eye8, jnp.ones((M, DH), _f32))
    pexp2 = jnp.kron(eye8, proj2.T)
    pexp3 = jnp.kron(eye8, proj3.T)
    ones16 = jnp.ones((CHR, 16), _f32)
    z32 = jnp.zeros((PZ, D), _f32)
    z16 = jnp.zeros((PZ, 16), _f32)

    p, xr = _tck_prep(x, wcat, W_root, b_root.reshape(1, D))
    agg2 = _sc_spline(p, src, dst, u0, u1, u2, z32)
    deg2 = _sc_deg(dst, ones16, z16)
    x1 = _tck_post1(agg2, deg2, xr, bn1_g.reshape(1, D), bn1_b.reshape(1, D))

    def layer(xin, wq, wk, wv, wo, pexp, g, b, use_lrelu):
        phiq, v, kv, sp = _tck_qkv(xin, wq, wk, wv, pexp, smask)
        rel = _sc_rel(v, src, dst, z32)
        attn = _tck_attn(phiq, kv, sp, bmask)
        return _tck_post(attn, rel, deg2, wo, g.reshape(1, D),
                         b.reshape(1, D), use_lrelu)

    x2 = layer(x1, Wq2, Wk2, Wv2, Wo2, pexp2, bn2_g, bn2_b, True)
    x3 = layer(x2, Wq3, Wk3, Wv3, Wo3, pexp3, bn3_g, bn3_b, False)
    return x3
